# baseline (speedup 1.0000x reference)
import functools

import jax
import jax.numpy as jnp
from jax import lax
from jax.experimental import pallas as pl
from jax.experimental.pallas import tpu as pltpu
from jax.experimental.pallas import tpu_sc as plsc

N_NODES = 10000
N_EDGES = 160000
N_WORKERS = 32
ROWS_PER_SUB = 640
N_PAD = ROWS_PER_SUB * 16
E_PAD = 163840
E_PER_W = E_PAD // N_WORKERS
CHUNK = 128
N_CHUNKS = E_PER_W // CHUNK
BM = 1000


def _sc_mesh():
    return plsc.VectorSubcoreMesh(core_axis_name="c", subcore_axis_name="s")


def _make_scatter_kernel(n_halves):
    out_type = [jax.ShapeDtypeStruct((2, N_NODES, 128), jnp.float32)
                for _ in range(n_halves)]
    scratch = [
        pltpu.VMEM((N_CHUNKS, CHUNK), jnp.int32),
        pltpu.VMEM((N_CHUNKS, CHUNK), jnp.int32),
        pltpu.VMEM((2, CHUNK, 128), jnp.float32),
        pltpu.VMEM_SHARED((N_PAD, 128), jnp.float32),
        pltpu.SemaphoreType.DMA,
    ]
    TAIL = N_NODES - 15 * ROWS_PER_SUB

    @functools.partial(pl.kernel, mesh=_sc_mesh(), out_type=out_type,
                       scratch_types=scratch)
    def sc_kernel(*refs):
        g_hbms = refs[:n_halves]
        src_hbm, dst_hbm, zeros_hbm = refs[n_halves:n_halves + 3]
        out_hbms = refs[n_halves + 3:2 * n_halves + 3]
        sidx_v, didx_v, rows_v, acc, sem = refs[2 * n_halves + 3:]

        c = lax.axis_index("c")
        t = lax.axis_index("s")
        wid = c * 16 + t
        base = t * ROWS_PER_SUB

        pltpu.sync_copy(src_hbm.at[pl.ds(wid * N_CHUNKS, N_CHUNKS)], sidx_v)
        pltpu.sync_copy(dst_hbm.at[pl.ds(wid * N_CHUNKS, N_CHUNKS)], didx_v)

        for h in range(n_halves):
            g_hbm = g_hbms[h]
            out_hbm = out_hbms[h]

            @pl.when(jnp.logical_and(c == 0, t != 15))
            def _():
                pltpu.sync_copy(g_hbm.at[pl.ds(base, ROWS_PER_SUB)],
                                acc.at[pl.ds(base, ROWS_PER_SUB)])

            @pl.when(jnp.logical_and(c == 0, t == 15))
            def _():
                pltpu.sync_copy(g_hbm.at[pl.ds(15 * ROWS_PER_SUB, TAIL)],
                                acc.at[pl.ds(15 * ROWS_PER_SUB, TAIL)])

            @pl.when(jnp.logical_and(c == 1, t != 15))
            def _():
                pltpu.sync_copy(zeros_hbm.at[pl.ds(base, ROWS_PER_SUB)],
                                acc.at[pl.ds(base, ROWS_PER_SUB)])

            @pl.when(jnp.logical_and(c == 1, t == 15))
            def _():
                pltpu.sync_copy(zeros_hbm.at[pl.ds(15 * ROWS_PER_SUB, TAIL)],
                                acc.at[pl.ds(15 * ROWS_PER_SUB, TAIL)])

            plsc.subcore_barrier()

            def body2(j, carry):
                pltpu.async_copy(g_hbm.at[sidx_v.at[j]],
                                 rows_v.at[0], sem).wait()
                pltpu.sync_copy(rows_v.at[0], acc.at[didx_v.at[j]],
                                add=True)
                return carry

            lax.fori_loop(0, N_CHUNKS, body2, 0)
            plsc.subcore_barrier()

            @pl.when(t != 15)
            def _():
                pltpu.sync_copy(acc.at[pl.ds(base, ROWS_PER_SUB)],
                                out_hbm.at[c, pl.ds(base, ROWS_PER_SUB)])

            @pl.when(t == 15)
            def _():
                pltpu.sync_copy(acc.at[pl.ds(15 * ROWS_PER_SUB, TAIL)],
                                out_hbm.at[c, pl.ds(15 * ROWS_PER_SUB, TAIL)])

            plsc.subcore_barrier()

    return sc_kernel


def _bspec(shape, blocked_m=True):
    if blocked_m:
        return pl.BlockSpec(shape, lambda i: (i,) + (0,) * (len(shape) - 1))
    return pl.BlockSpec(shape, lambda i: (0,) * len(shape))


def _make_tc_first():
    def body(x_ref, w_ref, da_ref, db_ref, o0_ref, o1_ref):
        s = lax.rsqrt(da_ref[...] + db_ref[...])
        g = jnp.dot(x_ref[...], w_ref[...],
                    preferred_element_type=jnp.float32) * s
        o0_ref[...] = g[:, :128]
        o1_ref[...] = g[:, 128:]

    return pl.pallas_call(
        body,
        grid=(N_NODES // BM,),
        in_specs=[
            _bspec((BM, 256)),
            _bspec((256, 256), blocked_m=False),
            _bspec((BM, 1)),
            _bspec((BM, 1)),
        ],
        out_specs=[_bspec((BM, 128)), _bspec((BM, 128))],
        out_shape=[jax.ShapeDtypeStruct((N_NODES, 128), jnp.float32)] * 2,
    )


def _make_tc_step(din, dout, relu):
    h_in, h_out = din // 128, dout // 128

    def body(*refs):
        p_refs = refs[:2 * h_in]
        da_ref, db_ref, b_ref, w_ref = refs[2 * h_in:2 * h_in + 4]
        o_refs = refs[2 * h_in + 4:]
        s = lax.rsqrt(da_ref[...] + db_ref[...])
        cols = [s * (p_refs[2 * h][...] + p_refs[2 * h + 1][...])
                for h in range(h_in)]
        t = jnp.concatenate(cols, axis=1) + b_ref[...]
        if relu:
            t = jnp.maximum(t, 0.0)
        g = jnp.dot(t, w_ref[...], preferred_element_type=jnp.float32) * s
        for h in range(h_out):
            o_refs[h][...] = g[:, h * 128:(h + 1) * 128]

    return pl.pallas_call(
        body,
        grid=(N_NODES // BM,),
        in_specs=(
            [_bspec((BM, 128))] * (2 * h_in)
            + [_bspec((BM, 1)), _bspec((BM, 1))]
            + [_bspec((1, din), blocked_m=False),
               _bspec((din, dout), blocked_m=False)]
        ),
        out_specs=[_bspec((BM, 128))] * h_out,
        out_shape=[jax.ShapeDtypeStruct((N_NODES, 128), jnp.float32)] * h_out,
    )


def _make_tc_final():
    def body(p0_ref, p1_ref, p2_ref, p3_ref, da_ref, db_ref, b_ref, o_ref):
        s = lax.rsqrt(da_ref[...] + db_ref[...])
        o_ref[...] = jnp.concatenate(
            [s * (p0_ref[...] + p1_ref[...]),
             s * (p2_ref[...] + p3_ref[...])], axis=1) + b_ref[...]

    return pl.pallas_call(
        body,
        grid=(N_NODES // BM,),
        in_specs=(
            [_bspec((BM, 128))] * 4
            + [_bspec((BM, 1)), _bspec((BM, 1))]
            + [_bspec((1, 256), blocked_m=False)]
        ),
        out_specs=_bspec((BM, 256)),
        out_shape=jax.ShapeDtypeStruct((N_NODES, 256), jnp.float32),
    )


def kernel(x, edge_index, W1, b1, W2, b2, W3, b3, W4, b4, W5, b5, W6, b6):
    src = edge_index[0].astype(jnp.int32)
    dst = edge_index[1].astype(jnp.int32)
    e_pad = E_PAD - src.shape[0]
    src_p = jnp.concatenate([src, jnp.zeros((e_pad,), jnp.int32)])
    dst_p = jnp.concatenate([dst, jnp.full((e_pad,), N_NODES, jnp.int32)])
    src_p = src_p.reshape(E_PAD // CHUNK, CHUNK)
    dst_p = dst_p.reshape(E_PAD // CHUNK, CHUNK)
    src_z = jnp.zeros_like(src_p)

    ones128 = jnp.ones((N_NODES, 128), jnp.float32)
    zeros128 = jnp.zeros((N_NODES, 128), jnp.float32)

    sc1 = _make_scatter_kernel(1)
    sc2 = _make_scatter_kernel(2)
    tc_first = _make_tc_first()
    tc_256_256 = _make_tc_step(256, 256, relu=True)
    tc_256_128 = _make_tc_step(256, 128, relu=True)
    tc_128_256 = _make_tc_step(128, 256, relu=False)
    tc_final = _make_tc_final()

    (deg,) = sc1(ones128, src_z, dst_p, zeros128)
    da = deg[0, :, 0:1]
    db = deg[1, :, 0:1]

    g0, g1 = tc_first(x, W1, da, db)
    p = sc2(g0, g1, src_p, dst_p, zeros128)
    g0, g1 = tc_256_256(p[0][0], p[0][1], p[1][0], p[1][1], da, db,
                        b1.reshape(1, -1), W2)
    p = sc2(g0, g1, src_p, dst_p, zeros128)
    (g0,) = tc_256_128(p[0][0], p[0][1], p[1][0], p[1][1], da, db,
                       b2.reshape(1, -1), W3)
    p = sc1(g0, src_p, dst_p, zeros128)
    g0, g1 = tc_128_256(p[0][0], p[0][1], da, db, b3.reshape(1, -1), W4)
    p = sc2(g0, g1, src_p, dst_p, zeros128)
    g0, g1 = tc_256_256(p[0][0], p[0][1], p[1][0], p[1][1], da, db,
                        b4.reshape(1, -1), W5)
    p = sc2(g0, g1, src_p, dst_p, zeros128)
    g0, g1 = tc_256_256(p[0][0], p[0][1], p[1][0], p[1][1], da, db,
                        b5.reshape(1, -1), W6)
    p = sc2(g0, g1, src_p, dst_p, zeros128)
    out = tc_final(p[0][0], p[0][1], p[1][0], p[1][1], da, db,
                   b6.reshape(1, -1))
    return out


# speedup vs baseline: 1.0613x; 1.0613x over previous
import functools

import jax
import jax.numpy as jnp
from jax import lax
from jax.experimental import pallas as pl
from jax.experimental.pallas import tpu as pltpu
from jax.experimental.pallas import tpu_sc as plsc

N_NODES = 10000
N_EDGES = 160000
N_WORKERS = 32
ROWS_PER_SUB = 640
N_PAD = ROWS_PER_SUB * 16
E_PAD = 163840
E_PER_W = E_PAD // N_WORKERS
CHUNK = 128
N_CHUNKS = E_PER_W // CHUNK
BM = 1000


def _sc_mesh():
    return plsc.VectorSubcoreMesh(core_axis_name="c", subcore_axis_name="s")


def _make_scatter_kernel(n_halves):
    out_type = [jax.ShapeDtypeStruct((2, N_NODES, 128), jnp.float32)
                for _ in range(n_halves)]
    scratch = [
        pltpu.VMEM((CHUNK,), jnp.int32),
        pltpu.VMEM((CHUNK,), jnp.int32),
        pltpu.VMEM((CHUNK,), jnp.int32),
        pltpu.VMEM((CHUNK,), jnp.int32),
        pltpu.VMEM((CHUNK, 128), jnp.float32),
        pltpu.VMEM((CHUNK, 128), jnp.float32),
        pltpu.VMEM_SHARED((N_PAD, 128), jnp.float32),
        pltpu.SemaphoreType.DMA,
        pltpu.SemaphoreType.DMA,
    ]
    TAIL = N_NODES - 15 * ROWS_PER_SUB

    @functools.partial(pl.kernel, mesh=_sc_mesh(), out_type=out_type,
                       scratch_types=scratch)
    def sc_kernel(*refs):
        g_hbms = refs[:n_halves]
        src_hbm, dst_hbm, zeros_hbm = refs[n_halves:n_halves + 3]
        out_hbms = refs[n_halves + 3:2 * n_halves + 3]
        (sidx_a, sidx_b, didx_a, didx_b, rows_a, rows_b,
         acc, sem_a, sem_b) = refs[2 * n_halves + 3:]
        sidx = (sidx_a, sidx_b)
        didx = (didx_a, didx_b)
        rows = (rows_a, rows_b)
        sems = (sem_a, sem_b)

        c = lax.axis_index("c")
        t = lax.axis_index("s")
        wid = c * 16 + t
        base = t * ROWS_PER_SUB
        ebase0 = wid * E_PER_W

        for h in range(n_halves):
            g_hbm = g_hbms[h]
            out_hbm = out_hbms[h]

            @pl.when(jnp.logical_and(c == 0, t != 15))
            def _():
                pltpu.sync_copy(g_hbm.at[pl.ds(base, ROWS_PER_SUB)],
                                acc.at[pl.ds(base, ROWS_PER_SUB)])

            @pl.when(jnp.logical_and(c == 0, t == 15))
            def _():
                pltpu.sync_copy(g_hbm.at[pl.ds(15 * ROWS_PER_SUB, TAIL)],
                                acc.at[pl.ds(15 * ROWS_PER_SUB, TAIL)])

            @pl.when(jnp.logical_and(c == 1, t != 15))
            def _():
                pltpu.sync_copy(zeros_hbm.at[pl.ds(base, ROWS_PER_SUB)],
                                acc.at[pl.ds(base, ROWS_PER_SUB)])

            @pl.when(jnp.logical_and(c == 1, t == 15))
            def _():
                pltpu.sync_copy(zeros_hbm.at[pl.ds(15 * ROWS_PER_SUB, TAIL)],
                                acc.at[pl.ds(15 * ROWS_PER_SUB, TAIL)])

            plsc.subcore_barrier()

            pltpu.sync_copy(src_hbm.at[pl.ds(ebase0, CHUNK)], sidx[0])
            pltpu.async_copy(g_hbm.at[sidx[0]], rows[0], sems[0])

            def body2(j2, carry):
                for b in range(2):
                    j = j2 * 2 + b
                    nb = 1 - b

                    @pl.when(j < N_CHUNKS - 1)
                    def _():
                        eb = pl.multiple_of(ebase0 + (j + 1) * CHUNK, CHUNK)
                        pltpu.sync_copy(src_hbm.at[pl.ds(eb, CHUNK)],
                                        sidx[nb])
                        pltpu.async_copy(g_hbm.at[sidx[nb]], rows[nb],
                                         sems[nb])

                    eb = pl.multiple_of(ebase0 + j * CHUNK, CHUNK)
                    pltpu.sync_copy(dst_hbm.at[pl.ds(eb, CHUNK)], didx[b])
                    pltpu.make_async_copy(g_hbm.at[sidx[b]], rows[b],
                                          sems[b]).wait()
                    pltpu.sync_copy(rows[b], acc.at[didx[b]], add=True)
                return carry

            lax.fori_loop(0, N_CHUNKS // 2, body2, 0)
            plsc.subcore_barrier()

            @pl.when(t != 15)
            def _():
                pltpu.sync_copy(acc.at[pl.ds(base, ROWS_PER_SUB)],
                                out_hbm.at[c, pl.ds(base, ROWS_PER_SUB)])

            @pl.when(t == 15)
            def _():
                pltpu.sync_copy(acc.at[pl.ds(15 * ROWS_PER_SUB, TAIL)],
                                out_hbm.at[c, pl.ds(15 * ROWS_PER_SUB, TAIL)])

            plsc.subcore_barrier()

    return sc_kernel


def _bspec(shape, blocked_m=True):
    if blocked_m:
        return pl.BlockSpec(shape, lambda i: (i,) + (0,) * (len(shape) - 1))
    return pl.BlockSpec(shape, lambda i: (0,) * len(shape))


def _make_tc_first():
    def body(x_ref, w_ref, da_ref, db_ref, o0_ref, o1_ref):
        s = lax.rsqrt(da_ref[...] + db_ref[...])
        g = jnp.dot(x_ref[...], w_ref[...],
                    preferred_element_type=jnp.float32) * s
        o0_ref[...] = g[:, :128]
        o1_ref[...] = g[:, 128:]

    return pl.pallas_call(
        body,
        grid=(N_NODES // BM,),
        in_specs=[
            _bspec((BM, 256)),
            _bspec((256, 256), blocked_m=False),
            _bspec((BM, 1)),
            _bspec((BM, 1)),
        ],
        out_specs=[_bspec((BM, 128)), _bspec((BM, 128))],
        out_shape=[jax.ShapeDtypeStruct((N_NODES, 128), jnp.float32)] * 2,
    )


def _make_tc_step(din, dout, relu):
    h_in, h_out = din // 128, dout // 128

    def body(*refs):
        p_refs = refs[:2 * h_in]
        da_ref, db_ref, b_ref, w_ref = refs[2 * h_in:2 * h_in + 4]
        o_refs = refs[2 * h_in + 4:]
        s = lax.rsqrt(da_ref[...] + db_ref[...])
        cols = [s * (p_refs[2 * h][...] + p_refs[2 * h + 1][...])
                for h in range(h_in)]
        t = jnp.concatenate(cols, axis=1) + b_ref[...]
        if relu:
            t = jnp.maximum(t, 0.0)
        g = jnp.dot(t, w_ref[...], preferred_element_type=jnp.float32) * s
        for h in range(h_out):
            o_refs[h][...] = g[:, h * 128:(h + 1) * 128]

    return pl.pallas_call(
        body,
        grid=(N_NODES // BM,),
        in_specs=(
            [_bspec((BM, 128))] * (2 * h_in)
            + [_bspec((BM, 1)), _bspec((BM, 1))]
            + [_bspec((1, din), blocked_m=False),
               _bspec((din, dout), blocked_m=False)]
        ),
        out_specs=[_bspec((BM, 128))] * h_out,
        out_shape=[jax.ShapeDtypeStruct((N_NODES, 128), jnp.float32)] * h_out,
    )


def _make_tc_final():
    def body(p0_ref, p1_ref, p2_ref, p3_ref, da_ref, db_ref, b_ref, o_ref):
        s = lax.rsqrt(da_ref[...] + db_ref[...])
        o_ref[...] = jnp.concatenate(
            [s * (p0_ref[...] + p1_ref[...]),
             s * (p2_ref[...] + p3_ref[...])], axis=1) + b_ref[...]

    return pl.pallas_call(
        body,
        grid=(N_NODES // BM,),
        in_specs=(
            [_bspec((BM, 128))] * 4
            + [_bspec((BM, 1)), _bspec((BM, 1))]
            + [_bspec((1, 256), blocked_m=False)]
        ),
        out_specs=_bspec((BM, 256)),
        out_shape=jax.ShapeDtypeStruct((N_NODES, 256), jnp.float32),
    )


def kernel(x, edge_index, W1, b1, W2, b2, W3, b3, W4, b4, W5, b5, W6, b6):
    src = edge_index[0].astype(jnp.int32)
    dst = edge_index[1].astype(jnp.int32)
    e_pad = E_PAD - src.shape[0]
    src_p = jnp.concatenate([src, jnp.zeros((e_pad,), jnp.int32)])
    dst_p = jnp.concatenate([dst, jnp.full((e_pad,), N_NODES, jnp.int32)])
    src_z = jnp.zeros_like(src_p)

    ones128 = jnp.ones((N_NODES, 128), jnp.float32)
    zeros128 = jnp.zeros((N_NODES, 128), jnp.float32)

    sc1 = _make_scatter_kernel(1)
    sc2 = _make_scatter_kernel(2)
    tc_first = _make_tc_first()
    tc_256_256 = _make_tc_step(256, 256, relu=True)
    tc_256_128 = _make_tc_step(256, 128, relu=True)
    tc_128_256 = _make_tc_step(128, 256, relu=False)
    tc_final = _make_tc_final()

    (deg,) = sc1(ones128, src_z, dst_p, zeros128)
    da = deg[0, :, 0:1]
    db = deg[1, :, 0:1]

    g0, g1 = tc_first(x, W1, da, db)
    p = sc2(g0, g1, src_p, dst_p, zeros128)
    g0, g1 = tc_256_256(p[0][0], p[0][1], p[1][0], p[1][1], da, db,
                        b1.reshape(1, -1), W2)
    p = sc2(g0, g1, src_p, dst_p, zeros128)
    (g0,) = tc_256_128(p[0][0], p[0][1], p[1][0], p[1][1], da, db,
                       b2.reshape(1, -1), W3)
    p = sc1(g0, src_p, dst_p, zeros128)
    g0, g1 = tc_128_256(p[0][0], p[0][1], da, db, b3.reshape(1, -1), W4)
    p = sc2(g0, g1, src_p, dst_p, zeros128)
    g0, g1 = tc_256_256(p[0][0], p[0][1], p[1][0], p[1][1], da, db,
                        b4.reshape(1, -1), W5)
    p = sc2(g0, g1, src_p, dst_p, zeros128)
    g0, g1 = tc_256_256(p[0][0], p[0][1], p[1][0], p[1][1], da, db,
                        b5.reshape(1, -1), W6)
    p = sc2(g0, g1, src_p, dst_p, zeros128)
    out = tc_final(p[0][0], p[0][1], p[1][0], p[1][1], da, db,
                   b6.reshape(1, -1))
    return out


# speedup vs baseline: 3.1064x; 2.9270x over previous
import functools

import jax
import jax.numpy as jnp
from jax import lax
from jax.experimental import pallas as pl
from jax.experimental.pallas import tpu as pltpu
from jax.experimental.pallas import tpu_sc as plsc

N_NODES = 10000
N_EDGES = 160000
N_WORKERS = 32
ROWS_PER_SUB = 640
N_PAD = ROWS_PER_SUB * 16
E_PAD = 163840
E_PER_W = E_PAD // N_WORKERS
CHUNK = 128
N_CHUNKS = E_PER_W // CHUNK
BM = 1000


def _sc_mesh():
    return plsc.VectorSubcoreMesh(core_axis_name="c", subcore_axis_name="s")


def _make_scatter_kernel(n_halves):
    out_type = [jax.ShapeDtypeStruct((2, N_NODES, 128), jnp.float32)
                for _ in range(n_halves)]
    scratch = [
        pltpu.VMEM((CHUNK,), jnp.int32),
        pltpu.VMEM((CHUNK,), jnp.int32),
        pltpu.VMEM((CHUNK,), jnp.int32),
        pltpu.VMEM((CHUNK,), jnp.int32),
        pltpu.VMEM((CHUNK, 128), jnp.float32),
        pltpu.VMEM((CHUNK, 128), jnp.float32),
        pltpu.VMEM_SHARED((N_PAD, 128), jnp.float32),
        pltpu.SemaphoreType.DMA,
        pltpu.SemaphoreType.DMA,
    ]
    TAIL = N_NODES - 15 * ROWS_PER_SUB

    @functools.partial(pl.kernel, mesh=_sc_mesh(), out_type=out_type,
                       scratch_types=scratch)
    def sc_kernel(*refs):
        g_hbms = refs[:n_halves]
        src_hbm, dst_hbm, zeros_hbm = refs[n_halves:n_halves + 3]
        out_hbms = refs[n_halves + 3:2 * n_halves + 3]
        (sidx_a, sidx_b, didx_a, didx_b, rows_a, rows_b,
         acc, sem_a, sem_b) = refs[2 * n_halves + 3:]
        sidx = (sidx_a, sidx_b)
        didx = (didx_a, didx_b)
        rows = (rows_a, rows_b)
        sems = (sem_a, sem_b)

        c = lax.axis_index("c")
        t = lax.axis_index("s")
        wid = c * 16 + t
        base = t * ROWS_PER_SUB
        ebase0 = wid * E_PER_W

        for h in range(n_halves):
            g_hbm = g_hbms[h]
            out_hbm = out_hbms[h]

            @pl.when(jnp.logical_and(c == 0, t != 15))
            def _():
                pltpu.sync_copy(g_hbm.at[pl.ds(base, ROWS_PER_SUB)],
                                acc.at[pl.ds(base, ROWS_PER_SUB)])

            @pl.when(jnp.logical_and(c == 0, t == 15))
            def _():
                pltpu.sync_copy(g_hbm.at[pl.ds(15 * ROWS_PER_SUB, TAIL)],
                                acc.at[pl.ds(15 * ROWS_PER_SUB, TAIL)])

            @pl.when(jnp.logical_and(c == 1, t != 15))
            def _():
                pltpu.sync_copy(zeros_hbm.at[pl.ds(base, ROWS_PER_SUB)],
                                acc.at[pl.ds(base, ROWS_PER_SUB)])

            @pl.when(jnp.logical_and(c == 1, t == 15))
            def _():
                pltpu.sync_copy(zeros_hbm.at[pl.ds(15 * ROWS_PER_SUB, TAIL)],
                                acc.at[pl.ds(15 * ROWS_PER_SUB, TAIL)])

            plsc.subcore_barrier()

            pltpu.sync_copy(src_hbm.at[pl.ds(ebase0, CHUNK)], sidx[0])
            pltpu.async_copy(g_hbm.at[sidx[0]], rows[0], sems[0])

            def body2(j2, carry):
                for b in range(2):
                    j = j2 * 2 + b
                    nb = 1 - b

                    @pl.when(j < N_CHUNKS - 1)
                    def _():
                        eb = pl.multiple_of(ebase0 + (j + 1) * CHUNK, CHUNK)
                        pltpu.sync_copy(src_hbm.at[pl.ds(eb, CHUNK)],
                                        sidx[nb])
                        pltpu.async_copy(g_hbm.at[sidx[nb]], rows[nb],
                                         sems[nb])

                    eb = pl.multiple_of(ebase0 + j * CHUNK, CHUNK)
                    pltpu.sync_copy(dst_hbm.at[pl.ds(eb, CHUNK)], didx[b])
                    pltpu.make_async_copy(g_hbm.at[sidx[b]], rows[b],
                                          sems[b]).wait()
                    pltpu.sync_copy(rows[b], acc.at[didx[b]], add=True)
                return carry

            lax.fori_loop(0, N_CHUNKS // 2, body2, 0)
            plsc.subcore_barrier()

            @pl.when(t != 15)
            def _():
                pltpu.sync_copy(acc.at[pl.ds(base, ROWS_PER_SUB)],
                                out_hbm.at[c, pl.ds(base, ROWS_PER_SUB)])

            @pl.when(t == 15)
            def _():
                pltpu.sync_copy(acc.at[pl.ds(15 * ROWS_PER_SUB, TAIL)],
                                out_hbm.at[c, pl.ds(15 * ROWS_PER_SUB, TAIL)])

            plsc.subcore_barrier()

    return sc_kernel


def _bspec(shape, blocked_m=True):
    if blocked_m:
        return pl.BlockSpec(shape, lambda i: (i,) + (0,) * (len(shape) - 1))
    return pl.BlockSpec(shape, lambda i: (0,) * len(shape))


def _make_tc_first():
    def body(x_ref, w_ref, da_ref, db_ref, o0_ref, o1_ref):
        s = lax.rsqrt(da_ref[...] + db_ref[...])
        g = jnp.dot(x_ref[...], w_ref[...],
                    preferred_element_type=jnp.float32) * s
        o0_ref[...] = g[:, :128]
        o1_ref[...] = g[:, 128:]

    return pl.pallas_call(
        body,
        grid=(N_NODES // BM,),
        in_specs=[
            _bspec((BM, 256)),
            _bspec((256, 256), blocked_m=False),
            _bspec((BM, 1)),
            _bspec((BM, 1)),
        ],
        out_specs=[_bspec((BM, 128)), _bspec((BM, 128))],
        out_shape=[jax.ShapeDtypeStruct((N_NODES, 128), jnp.float32)] * 2,
    )


def _make_tc_step(din, dout, relu):
    h_in, h_out = din // 128, dout // 128

    def body(*refs):
        p_refs = refs[:2 * h_in]
        da_ref, db_ref, b_ref, w_ref = refs[2 * h_in:2 * h_in + 4]
        o_refs = refs[2 * h_in + 4:]
        s = lax.rsqrt(da_ref[...] + db_ref[...])
        cols = [s * (p_refs[2 * h][...] + p_refs[2 * h + 1][...])
                for h in range(h_in)]
        t = jnp.concatenate(cols, axis=1) + b_ref[...]
        if relu:
            t = jnp.maximum(t, 0.0)
        g = jnp.dot(t, w_ref[...], preferred_element_type=jnp.float32) * s
        for h in range(h_out):
            o_refs[h][...] = g[:, h * 128:(h + 1) * 128]

    return pl.pallas_call(
        body,
        grid=(N_NODES // BM,),
        in_specs=(
            [_bspec((BM, 128))] * (2 * h_in)
            + [_bspec((BM, 1)), _bspec((BM, 1))]
            + [_bspec((1, din), blocked_m=False),
               _bspec((din, dout), blocked_m=False)]
        ),
        out_specs=[_bspec((BM, 128))] * h_out,
        out_shape=[jax.ShapeDtypeStruct((N_NODES, 128), jnp.float32)] * h_out,
    )


def _make_tc_final():
    def body(p0_ref, p1_ref, p2_ref, p3_ref, da_ref, db_ref, b_ref, o_ref):
        s = lax.rsqrt(da_ref[...] + db_ref[...])
        o_ref[...] = jnp.concatenate(
            [s * (p0_ref[...] + p1_ref[...]),
             s * (p2_ref[...] + p3_ref[...])], axis=1) + b_ref[...]

    return pl.pallas_call(
        body,
        grid=(N_NODES // BM,),
        in_specs=(
            [_bspec((BM, 128))] * 4
            + [_bspec((BM, 1)), _bspec((BM, 1))]
            + [_bspec((1, 256), blocked_m=False)]
        ),
        out_specs=_bspec((BM, 256)),
        out_shape=jax.ShapeDtypeStruct((N_NODES, 256), jnp.float32),
    )


def kernel(x, edge_index, W1, b1, W2, b2, W3, b3, W4, b4, W5, b5, W6, b6):
    src = edge_index[0].astype(jnp.int32)
    dst = edge_index[1].astype(jnp.int32)
    e_pad = E_PAD - src.shape[0]
    src_p = jnp.concatenate([src, jnp.zeros((e_pad,), jnp.int32)])
    dst_p = jnp.concatenate([dst, jnp.full((e_pad,), N_NODES, jnp.int32)])

    ones128 = jnp.ones((N_NODES, 128), jnp.float32)
    zeros128 = jnp.zeros((N_NODES, 128), jnp.float32)

    sc1 = _make_scatter_kernel(1)
    sc2 = _make_scatter_kernel(2)
    tc_first = _make_tc_first()
    tc_256_256 = _make_tc_step(256, 256, relu=True)
    tc_256_128 = _make_tc_step(256, 128, relu=True)
    tc_128_256 = _make_tc_step(128, 256, relu=False)
    tc_final = _make_tc_final()

    (deg,) = sc1(ones128, src_p, dst_p, zeros128)
    da = deg[0, :, 0:1]
    db = deg[1, :, 0:1]

    g0, g1 = tc_first(x, W1, da, db)
    p = sc2(g0, g1, src_p, dst_p, zeros128)
    g0, g1 = tc_256_256(p[0][0], p[0][1], p[1][0], p[1][1], da, db,
                        b1.reshape(1, -1), W2)
    p = sc2(g0, g1, src_p, dst_p, zeros128)
    (g0,) = tc_256_128(p[0][0], p[0][1], p[1][0], p[1][1], da, db,
                       b2.reshape(1, -1), W3)
    p = sc1(g0, src_p, dst_p, zeros128)
    g0, g1 = tc_128_256(p[0][0], p[0][1], da, db, b3.reshape(1, -1), W4)
    p = sc2(g0, g1, src_p, dst_p, zeros128)
    g0, g1 = tc_256_256(p[0][0], p[0][1], p[1][0], p[1][1], da, db,
                        b4.reshape(1, -1), W5)
    p = sc2(g0, g1, src_p, dst_p, zeros128)
    g0, g1 = tc_256_256(p[0][0], p[0][1], p[1][0], p[1][1], da, db,
                        b5.reshape(1, -1), W6)
    p = sc2(g0, g1, src_p, dst_p, zeros128)
    out = tc_final(p[0][0], p[0][1], p[1][0], p[1][1], da, db,
                   b6.reshape(1, -1))
    return out
